# Initial kernel scaffold; baseline (speedup 1.0000x reference)
import jax
import jax.numpy as jnp
from jax.experimental import pallas as pl


def kernel(a, W1, g1, b1, W2, g2, b2):
    raise NotImplementedError("write your pallas kernel here")


# speedup vs baseline: 3.0536x; 3.0536x over previous
import functools
import jax
import jax.numpy as jnp
from jax.experimental import pallas as pl
from jax.experimental.pallas import tpu as pltpu

KNN = 20
EPS = 1e-5


def _leaky(x):
    return jnp.where(x >= 0, x, 0.2 * x)


def _dot(a, b, dims, precision=jax.lax.Precision.HIGHEST):
    return jax.lax.dot_general(a, b, (dims, ((), ())),
                               precision=precision,
                               preferred_element_type=jnp.float32)


def _stage_a_body(a_all_ref, a_tile_ref, w1a_ref, w1d_ref,
                  nb_ref, p1_ref, d_ref, ssum_ref, ssq_ref, *, tn, n):
    a_all = a_all_ref[0]
    a_tile = a_tile_ref[0]
    w1a = w1a_ref[...]
    w1d = w1d_ref[...]

    sq = jnp.sum(a_all * a_all, axis=0, keepdims=True)
    xx = _dot(a_tile, a_all, ((0,), (0,)),
              precision=jax.lax.Precision.DEFAULT)
    d_ref[...] = sq - 2.0 * xx

    vt = _dot(w1d, a_tile, ((1,), (0,)))
    ssum_ref[...] = jnp.zeros((64, tn), jnp.float32)
    ssq_ref[...] = jnp.zeros((64, tn), jnp.float32)

    def step(k, _):
        d = d_ref[...]
        col = jax.lax.broadcasted_iota(jnp.int32, (tn, n), 1)
        m = jnp.min(d, axis=1, keepdims=True)
        cand = jnp.where(d <= m, col, n)
        jmin = jnp.min(cand, axis=1, keepdims=True)
        mask = cand == jmin
        maskf = mask.astype(jnp.float32)
        nbkt = _dot(a_all, maskf, ((1,), (1,)))
        nb_ref[0, pl.ds(k, 1)] = nbkt[None]
        d_ref[...] = jnp.where(mask, jnp.inf, d)
        h1 = _dot(w1a, nbkt, ((1,), (0,))) + vt
        ssum_ref[...] += h1
        ssq_ref[...] += h1 * h1
        return 0

    jax.lax.fori_loop(0, KNN, step, 0)
    p1_ref[0, 0] = jnp.concatenate(
        [jnp.sum(ssum_ref[...], axis=1, keepdims=True),
         jnp.sum(ssq_ref[...], axis=1, keepdims=True)], axis=1)


def _stage_c_body(nb_ref, a_tile_ref, w1a_ref, w1d_ref, w2_ref,
                  s1_ref, t1_ref, zmax_ref, zmin_ref, p2_ref,
                  zsum_ref, zsq_ref, *, tn):
    a_tile = a_tile_ref[0]
    w1a = w1a_ref[...]
    w1d = w1d_ref[...]
    w2 = w2_ref[...]
    s1 = s1_ref[...]
    t1 = t1_ref[...]

    vt = _dot(w1d, a_tile, ((1,), (0,)))

    zmax_ref[0] = jnp.full((64, tn), -jnp.inf, jnp.float32)
    zmin_ref[0] = jnp.full((64, tn), jnp.inf, jnp.float32)
    zsum_ref[...] = jnp.zeros((64, tn), jnp.float32)
    zsq_ref[...] = jnp.zeros((64, tn), jnp.float32)

    def step(k, _):
        nbkt = nb_ref[0, pl.ds(k, 1)][0]
        h1 = _dot(w1a, nbkt, ((1,), (0,))) + vt
        hp = _leaky(h1 * s1 + t1)
        z = _dot(w2, hp, ((1,), (0,)))
        zmax_ref[0] = jnp.maximum(zmax_ref[0], z)
        zmin_ref[0] = jnp.minimum(zmin_ref[0], z)
        zsum_ref[...] += z
        zsq_ref[...] += z * z
        return 0

    jax.lax.fori_loop(0, KNN, step, 0)
    p2_ref[0, 0] = jnp.concatenate(
        [jnp.sum(zsum_ref[...], axis=1, keepdims=True),
         jnp.sum(zsq_ref[...], axis=1, keepdims=True)], axis=1)


def _stage_d_body(zmax_ref, zmin_ref, s2_ref, t2_ref, out_ref):
    s2 = s2_ref[...]
    t2 = t2_ref[...]
    zsel = jnp.where(s2 >= 0, zmax_ref[0], zmin_ref[0])
    out_ref[0] = _leaky(s2 * zsel + t2)


def kernel(a, W1, g1, b1, W2, g2, b2):
    B, C, N = a.shape
    CO = W1.shape[0]
    TN = 256
    T = N // TN
    cnt = B * N * KNN

    w1a = W1[:, :C]
    w1d = W1[:, C:] - w1a

    grid = (B, T)
    nb, p1 = pl.pallas_call(
        functools.partial(_stage_a_body, tn=TN, n=N),
        grid=grid,
        in_specs=[
            pl.BlockSpec((1, C, N), lambda b, t: (b, 0, 0)),
            pl.BlockSpec((1, C, TN), lambda b, t: (b, 0, t)),
            pl.BlockSpec((CO, C), lambda b, t: (0, 0)),
            pl.BlockSpec((CO, C), lambda b, t: (0, 0)),
        ],
        out_specs=[
            pl.BlockSpec((1, KNN, C, TN), lambda b, t: (b, 0, 0, t)),
            pl.BlockSpec((1, 1, CO, 2), lambda b, t: (b, t, 0, 0)),
        ],
        out_shape=[
            jax.ShapeDtypeStruct((B, KNN, C, N), jnp.float32),
            jax.ShapeDtypeStruct((B, T, CO, 2), jnp.float32),
        ],
        scratch_shapes=[
            pltpu.VMEM((TN, N), jnp.float32),
            pltpu.VMEM((CO, TN), jnp.float32),
            pltpu.VMEM((CO, TN), jnp.float32),
        ],
    )(a, a, w1a, w1d)

    tot1 = jnp.sum(p1, axis=(0, 1))
    mean1 = tot1[:, 0] / cnt
    var1 = tot1[:, 1] / cnt - mean1 * mean1
    s1 = (g1 * jax.lax.rsqrt(var1 + EPS))[:, None]
    t1 = (b1 - mean1 * g1 * jax.lax.rsqrt(var1 + EPS))[:, None]

    zmax, zmin, p2 = pl.pallas_call(
        functools.partial(_stage_c_body, tn=TN),
        grid=grid,
        in_specs=[
            pl.BlockSpec((1, KNN, C, TN), lambda b, t: (b, 0, 0, t)),
            pl.BlockSpec((1, C, TN), lambda b, t: (b, 0, t)),
            pl.BlockSpec((CO, C), lambda b, t: (0, 0)),
            pl.BlockSpec((CO, C), lambda b, t: (0, 0)),
            pl.BlockSpec((CO, CO), lambda b, t: (0, 0)),
            pl.BlockSpec((CO, 1), lambda b, t: (0, 0)),
            pl.BlockSpec((CO, 1), lambda b, t: (0, 0)),
        ],
        out_specs=[
            pl.BlockSpec((1, CO, TN), lambda b, t: (b, 0, t)),
            pl.BlockSpec((1, CO, TN), lambda b, t: (b, 0, t)),
            pl.BlockSpec((1, 1, CO, 2), lambda b, t: (b, t, 0, 0)),
        ],
        out_shape=[
            jax.ShapeDtypeStruct((B, CO, N), jnp.float32),
            jax.ShapeDtypeStruct((B, CO, N), jnp.float32),
            jax.ShapeDtypeStruct((B, T, CO, 2), jnp.float32),
        ],
        scratch_shapes=[
            pltpu.VMEM((CO, TN), jnp.float32),
            pltpu.VMEM((CO, TN), jnp.float32),
        ],
    )(nb, a, w1a, w1d, W2, s1, t1)

    tot2 = jnp.sum(p2, axis=(0, 1))
    mean2 = tot2[:, 0] / cnt
    var2 = tot2[:, 1] / cnt - mean2 * mean2
    s2 = (g2 * jax.lax.rsqrt(var2 + EPS))[:, None]
    t2 = (b2 - mean2 * g2 * jax.lax.rsqrt(var2 + EPS))[:, None]

    out = pl.pallas_call(
        _stage_d_body,
        grid=grid,
        in_specs=[
            pl.BlockSpec((1, CO, TN), lambda b, t: (b, 0, t)),
            pl.BlockSpec((1, CO, TN), lambda b, t: (b, 0, t)),
            pl.BlockSpec((CO, 1), lambda b, t: (0, 0)),
            pl.BlockSpec((CO, 1), lambda b, t: (0, 0)),
        ],
        out_specs=pl.BlockSpec((1, CO, TN), lambda b, t: (b, 0, t)),
        out_shape=jax.ShapeDtypeStruct((B, CO, N), jnp.float32),
    )(zmax, zmin, s2, t2)
    return out


# speedup vs baseline: 7.5494x; 2.4723x over previous
import functools
import jax
import jax.numpy as jnp
from jax.experimental import pallas as pl
from jax.experimental.pallas import tpu as pltpu

KNN = 20
EPS = 1e-5


def _leaky(x):
    return jnp.where(x >= 0, x, 0.2 * x)


def _dot(a, b, dims, precision=jax.lax.Precision.HIGHEST):
    return jax.lax.dot_general(a, b, (dims, ((), ())),
                               precision=precision,
                               preferred_element_type=jnp.float32)


def _stage_a_body(a_all_ref, a_tile_ref, w1a_ref, w1d_ref,
                  nb_ref, p1_ref, d_ref, ssum_ref, ssq_ref, *, tn, n):
    a_all = a_all_ref[0]
    a_tile = a_tile_ref[0]
    w1a = w1a_ref[...]
    w1d = w1d_ref[...]

    sq = jnp.sum(a_all * a_all, axis=0, keepdims=True)
    xx = _dot(a_tile, a_all, ((0,), (0,)),
              precision=jax.lax.Precision.DEFAULT)
    dinit = sq - 2.0 * xx
    d_ref[...] = dinit
    m0 = jnp.min(dinit, axis=1, keepdims=True)

    a4 = jnp.concatenate([a_all, jnp.ones((1, n), jnp.float32)], axis=0)
    ahi = a4.astype(jnp.bfloat16)
    alo = (a4 - ahi.astype(jnp.float32)).astype(jnp.bfloat16)

    vt = _dot(w1d, a_tile, ((1,), (0,)))
    ssum_ref[...] = jnp.zeros((64, tn), jnp.float32)
    ssq_ref[...] = jnp.zeros((64, tn), jnp.float32)

    def step(k, m):
        d = d_ref[...]
        mask = d <= m
        maskb = mask.astype(jnp.bfloat16)
        dflt = jax.lax.Precision.DEFAULT
        s4 = (_dot(ahi, maskb, ((1,), (1,)), dflt)
              + _dot(alo, maskb, ((1,), (1,)), dflt))
        nbkt = s4[0:3, :] / s4[3:4, :]
        nb_ref[0, pl.ds(k, 1)] = nbkt[None]
        dn = jnp.where(mask, jnp.inf, d)
        d_ref[...] = dn
        h1 = _dot(w1a, nbkt, ((1,), (0,))) + vt
        ssum_ref[...] += h1
        ssq_ref[...] += h1 * h1
        return jnp.min(dn, axis=1, keepdims=True)

    jax.lax.fori_loop(0, KNN, step, m0)
    p1_ref[0, 0] = jnp.concatenate(
        [jnp.sum(ssum_ref[...], axis=1, keepdims=True),
         jnp.sum(ssq_ref[...], axis=1, keepdims=True)], axis=1)


def _stage_c_body(nb_ref, a_tile_ref, w1a_ref, w1d_ref, w2_ref,
                  s1_ref, t1_ref, zmax_ref, zmin_ref, p2_ref,
                  zsum_ref, zsq_ref, *, tn):
    a_tile = a_tile_ref[0]
    w1a = w1a_ref[...]
    w1d = w1d_ref[...]
    w2 = w2_ref[...]
    s1 = s1_ref[...]
    t1 = t1_ref[...]

    vt = _dot(w1d, a_tile, ((1,), (0,)))

    zmax_ref[0] = jnp.full((64, tn), -jnp.inf, jnp.float32)
    zmin_ref[0] = jnp.full((64, tn), jnp.inf, jnp.float32)
    zsum_ref[...] = jnp.zeros((64, tn), jnp.float32)
    zsq_ref[...] = jnp.zeros((64, tn), jnp.float32)

    def step(k, _):
        nbkt = nb_ref[0, pl.ds(k, 1)][0]
        h1 = _dot(w1a, nbkt, ((1,), (0,))) + vt
        hp = _leaky(h1 * s1 + t1)
        z = _dot(w2, hp, ((1,), (0,)))
        zmax_ref[0] = jnp.maximum(zmax_ref[0], z)
        zmin_ref[0] = jnp.minimum(zmin_ref[0], z)
        zsum_ref[...] += z
        zsq_ref[...] += z * z
        return 0

    jax.lax.fori_loop(0, KNN, step, 0)
    p2_ref[0, 0] = jnp.concatenate(
        [jnp.sum(zsum_ref[...], axis=1, keepdims=True),
         jnp.sum(zsq_ref[...], axis=1, keepdims=True)], axis=1)


def _stage_d_body(zmax_ref, zmin_ref, s2_ref, t2_ref, out_ref):
    s2 = s2_ref[...]
    t2 = t2_ref[...]
    zsel = jnp.where(s2 >= 0, zmax_ref[0], zmin_ref[0])
    out_ref[0] = _leaky(s2 * zsel + t2)


def kernel(a, W1, g1, b1, W2, g2, b2):
    B, C, N = a.shape
    CO = W1.shape[0]
    TN = 256
    T = N // TN
    cnt = B * N * KNN

    w1a = W1[:, :C]
    w1d = W1[:, C:] - w1a

    grid = (B, T)
    nb, p1 = pl.pallas_call(
        functools.partial(_stage_a_body, tn=TN, n=N),
        grid=grid,
        in_specs=[
            pl.BlockSpec((1, C, N), lambda b, t: (b, 0, 0)),
            pl.BlockSpec((1, C, TN), lambda b, t: (b, 0, t)),
            pl.BlockSpec((CO, C), lambda b, t: (0, 0)),
            pl.BlockSpec((CO, C), lambda b, t: (0, 0)),
        ],
        out_specs=[
            pl.BlockSpec((1, KNN, C, TN), lambda b, t: (b, 0, 0, t)),
            pl.BlockSpec((1, 1, CO, 2), lambda b, t: (b, t, 0, 0)),
        ],
        out_shape=[
            jax.ShapeDtypeStruct((B, KNN, C, N), jnp.float32),
            jax.ShapeDtypeStruct((B, T, CO, 2), jnp.float32),
        ],
        scratch_shapes=[
            pltpu.VMEM((TN, N), jnp.float32),
            pltpu.VMEM((CO, TN), jnp.float32),
            pltpu.VMEM((CO, TN), jnp.float32),
        ],
    )(a, a, w1a, w1d)

    tot1 = jnp.sum(p1, axis=(0, 1))
    mean1 = tot1[:, 0] / cnt
    var1 = tot1[:, 1] / cnt - mean1 * mean1
    s1 = (g1 * jax.lax.rsqrt(var1 + EPS))[:, None]
    t1 = (b1 - mean1 * g1 * jax.lax.rsqrt(var1 + EPS))[:, None]

    zmax, zmin, p2 = pl.pallas_call(
        functools.partial(_stage_c_body, tn=TN),
        grid=grid,
        in_specs=[
            pl.BlockSpec((1, KNN, C, TN), lambda b, t: (b, 0, 0, t)),
            pl.BlockSpec((1, C, TN), lambda b, t: (b, 0, t)),
            pl.BlockSpec((CO, C), lambda b, t: (0, 0)),
            pl.BlockSpec((CO, C), lambda b, t: (0, 0)),
            pl.BlockSpec((CO, CO), lambda b, t: (0, 0)),
            pl.BlockSpec((CO, 1), lambda b, t: (0, 0)),
            pl.BlockSpec((CO, 1), lambda b, t: (0, 0)),
        ],
        out_specs=[
            pl.BlockSpec((1, CO, TN), lambda b, t: (b, 0, t)),
            pl.BlockSpec((1, CO, TN), lambda b, t: (b, 0, t)),
            pl.BlockSpec((1, 1, CO, 2), lambda b, t: (b, t, 0, 0)),
        ],
        out_shape=[
            jax.ShapeDtypeStruct((B, CO, N), jnp.float32),
            jax.ShapeDtypeStruct((B, CO, N), jnp.float32),
            jax.ShapeDtypeStruct((B, T, CO, 2), jnp.float32),
        ],
        scratch_shapes=[
            pltpu.VMEM((CO, TN), jnp.float32),
            pltpu.VMEM((CO, TN), jnp.float32),
        ],
    )(nb, a, w1a, w1d, W2, s1, t1)

    tot2 = jnp.sum(p2, axis=(0, 1))
    mean2 = tot2[:, 0] / cnt
    var2 = tot2[:, 1] / cnt - mean2 * mean2
    s2 = (g2 * jax.lax.rsqrt(var2 + EPS))[:, None]
    t2 = (b2 - mean2 * g2 * jax.lax.rsqrt(var2 + EPS))[:, None]

    out = pl.pallas_call(
        _stage_d_body,
        grid=grid,
        in_specs=[
            pl.BlockSpec((1, CO, TN), lambda b, t: (b, 0, t)),
            pl.BlockSpec((1, CO, TN), lambda b, t: (b, 0, t)),
            pl.BlockSpec((CO, 1), lambda b, t: (0, 0)),
            pl.BlockSpec((CO, 1), lambda b, t: (0, 0)),
        ],
        out_specs=pl.BlockSpec((1, CO, TN), lambda b, t: (b, 0, t)),
        out_shape=jax.ShapeDtypeStruct((B, CO, N), jnp.float32),
    )(zmax, zmin, s2, t2)
    return out


# speedup vs baseline: 9.2851x; 1.2299x over previous
import functools
import jax
import jax.numpy as jnp
from jax.experimental import pallas as pl
from jax.experimental.pallas import tpu as pltpu

KNN = 20
EPS = 1e-5


def _leaky(x):
    return jnp.where(x >= 0, x, 0.2 * x)


def _dot(a, b, dims, precision=jax.lax.Precision.HIGHEST):
    return jax.lax.dot_general(a, b, (dims, ((), ())),
                               precision=precision,
                               preferred_element_type=jnp.float32)


def _stage_a_body(a_all_ref, a_tile_ref, w1a_ref, w1d_ref,
                  nb_ref, p1_ref, d_ref, ssum_ref, ssq_ref, *, tn, n):
    a_all = a_all_ref[0]
    a_tile = a_tile_ref[0]
    w1a = w1a_ref[...]
    w1d = w1d_ref[...]

    sq = jnp.sum(a_all * a_all, axis=0, keepdims=True)
    xx = _dot(a_tile, a_all, ((0,), (0,)),
              precision=jax.lax.Precision.DEFAULT)
    dinit = sq - 2.0 * xx
    d_ref[...] = dinit
    m0 = jnp.min(dinit, axis=1, keepdims=True)

    a4 = jnp.concatenate([a_all, jnp.ones((1, n), jnp.float32)], axis=0)
    ahi = a4.astype(jnp.bfloat16)
    alo = (a4 - ahi.astype(jnp.float32)).astype(jnp.bfloat16)

    vt = _dot(w1d, a_tile, ((1,), (0,)))
    ssum_ref[...] = jnp.zeros((64, tn), jnp.float32)
    ssq_ref[...] = jnp.zeros((64, tn), jnp.float32)

    def step(k, m):
        d = d_ref[...]
        mask = d <= m
        maskb = mask.astype(jnp.bfloat16)
        dflt = jax.lax.Precision.DEFAULT
        s4 = (_dot(ahi, maskb, ((1,), (1,)), dflt)
              + _dot(alo, maskb, ((1,), (1,)), dflt))
        nbkt = s4[0:3, :] / s4[3:4, :]
        nb_ref[0, pl.ds(k, 1)] = nbkt[None]
        dn = jnp.where(mask, jnp.inf, d)
        d_ref[...] = dn
        h1 = _dot(w1a, nbkt, ((1,), (0,))) + vt
        ssum_ref[...] += h1
        ssq_ref[...] += h1 * h1
        return jnp.min(dn, axis=1, keepdims=True)

    jax.lax.fori_loop(0, KNN, step, m0)
    p1_ref[0, 0] = jnp.concatenate(
        [jnp.sum(ssum_ref[...], axis=1, keepdims=True),
         jnp.sum(ssq_ref[...], axis=1, keepdims=True)], axis=1)


def _stage_c_body(nb_ref, a_tile_ref, w1a_ref, w1d_ref, w2_ref,
                  s1_ref, t1_ref, zmax_ref, zmin_ref, p2_ref,
                  zsum_ref, zsq_ref, *, tn):
    a_tile = a_tile_ref[0]
    w1a = w1a_ref[...]
    w1d = w1d_ref[...]
    w2 = w2_ref[...]
    s1 = s1_ref[...]
    t1 = t1_ref[...]

    dflt = jax.lax.Precision.DEFAULT
    vt = _dot(w1d, a_tile, ((1,), (0,)), dflt)

    zmax_ref[0] = jnp.full((64, tn), -jnp.inf, jnp.float32)
    zmin_ref[0] = jnp.full((64, tn), jnp.inf, jnp.float32)
    zsum_ref[...] = jnp.zeros((64, tn), jnp.float32)
    zsq_ref[...] = jnp.zeros((64, tn), jnp.float32)

    def step(k, _):
        nbkt = nb_ref[0, pl.ds(k, 1)][0]
        h1 = _dot(w1a, nbkt, ((1,), (0,)), dflt) + vt
        hp = _leaky(h1 * s1 + t1)
        z = _dot(w2, hp, ((1,), (0,)), dflt)
        zmax_ref[0] = jnp.maximum(zmax_ref[0], z)
        zmin_ref[0] = jnp.minimum(zmin_ref[0], z)
        zsum_ref[...] += z
        zsq_ref[...] += z * z
        return 0

    jax.lax.fori_loop(0, KNN, step, 0)
    p2_ref[0, 0] = jnp.concatenate(
        [jnp.sum(zsum_ref[...], axis=1, keepdims=True),
         jnp.sum(zsq_ref[...], axis=1, keepdims=True)], axis=1)


def _stage_d_body(zmax_ref, zmin_ref, s2_ref, t2_ref, out_ref):
    s2 = s2_ref[...]
    t2 = t2_ref[...]
    zsel = jnp.where(s2 >= 0, zmax_ref[0], zmin_ref[0])
    out_ref[0] = _leaky(s2 * zsel + t2)


def kernel(a, W1, g1, b1, W2, g2, b2):
    B, C, N = a.shape
    CO = W1.shape[0]
    TN = 256
    T = N // TN
    TNC = min(1024, N)
    TC_ = N // TNC
    cnt = B * N * KNN

    w1a = W1[:, :C]
    w1d = W1[:, C:] - w1a

    grid = (B, T)
    nb, p1 = pl.pallas_call(
        functools.partial(_stage_a_body, tn=TN, n=N),
        grid=grid,
        in_specs=[
            pl.BlockSpec((1, C, N), lambda b, t: (b, 0, 0)),
            pl.BlockSpec((1, C, TN), lambda b, t: (b, 0, t)),
            pl.BlockSpec((CO, C), lambda b, t: (0, 0)),
            pl.BlockSpec((CO, C), lambda b, t: (0, 0)),
        ],
        out_specs=[
            pl.BlockSpec((1, KNN, C, TN), lambda b, t: (b, 0, 0, t)),
            pl.BlockSpec((1, 1, CO, 2), lambda b, t: (b, t, 0, 0)),
        ],
        out_shape=[
            jax.ShapeDtypeStruct((B, KNN, C, N), jnp.float32),
            jax.ShapeDtypeStruct((B, T, CO, 2), jnp.float32),
        ],
        scratch_shapes=[
            pltpu.VMEM((TN, N), jnp.float32),
            pltpu.VMEM((CO, TN), jnp.float32),
            pltpu.VMEM((CO, TN), jnp.float32),
        ],
    )(a, a, w1a, w1d)

    tot1 = jnp.sum(p1, axis=(0, 1))
    mean1 = tot1[:, 0] / cnt
    var1 = tot1[:, 1] / cnt - mean1 * mean1
    s1 = (g1 * jax.lax.rsqrt(var1 + EPS))[:, None]
    t1 = (b1 - mean1 * g1 * jax.lax.rsqrt(var1 + EPS))[:, None]

    zmax, zmin, p2 = pl.pallas_call(
        functools.partial(_stage_c_body, tn=TNC),
        grid=(B, TC_),
        in_specs=[
            pl.BlockSpec((1, KNN, C, TNC), lambda b, t: (b, 0, 0, t)),
            pl.BlockSpec((1, C, TNC), lambda b, t: (b, 0, t)),
            pl.BlockSpec((CO, C), lambda b, t: (0, 0)),
            pl.BlockSpec((CO, C), lambda b, t: (0, 0)),
            pl.BlockSpec((CO, CO), lambda b, t: (0, 0)),
            pl.BlockSpec((CO, 1), lambda b, t: (0, 0)),
            pl.BlockSpec((CO, 1), lambda b, t: (0, 0)),
        ],
        out_specs=[
            pl.BlockSpec((1, CO, TNC), lambda b, t: (b, 0, t)),
            pl.BlockSpec((1, CO, TNC), lambda b, t: (b, 0, t)),
            pl.BlockSpec((1, 1, CO, 2), lambda b, t: (b, t, 0, 0)),
        ],
        out_shape=[
            jax.ShapeDtypeStruct((B, CO, N), jnp.float32),
            jax.ShapeDtypeStruct((B, CO, N), jnp.float32),
            jax.ShapeDtypeStruct((B, TC_, CO, 2), jnp.float32),
        ],
        scratch_shapes=[
            pltpu.VMEM((CO, TNC), jnp.float32),
            pltpu.VMEM((CO, TNC), jnp.float32),
        ],
    )(nb, a, w1a, w1d, W2, s1, t1)

    tot2 = jnp.sum(p2, axis=(0, 1))
    mean2 = tot2[:, 0] / cnt
    var2 = tot2[:, 1] / cnt - mean2 * mean2
    s2 = (g2 * jax.lax.rsqrt(var2 + EPS))[:, None]
    t2 = (b2 - mean2 * g2 * jax.lax.rsqrt(var2 + EPS))[:, None]

    out = pl.pallas_call(
        _stage_d_body,
        grid=(B, TC_),
        in_specs=[
            pl.BlockSpec((1, CO, TNC), lambda b, t: (b, 0, t)),
            pl.BlockSpec((1, CO, TNC), lambda b, t: (b, 0, t)),
            pl.BlockSpec((CO, 1), lambda b, t: (0, 0)),
            pl.BlockSpec((CO, 1), lambda b, t: (0, 0)),
        ],
        out_specs=pl.BlockSpec((1, CO, TNC), lambda b, t: (b, 0, t)),
        out_shape=jax.ShapeDtypeStruct((B, CO, N), jnp.float32),
    )(zmax, zmin, s2, t2)
    return out
